# baseline (speedup 1.0000x reference)
import functools

import jax
import jax.numpy as jnp
from jax import lax
from jax.experimental import pallas as pl
from jax.experimental.pallas import tpu as pltpu
from jax.experimental.pallas import tpu_sc as plsc

N = 10000
K = 16
D = 128
T = 3
EPS = 0.01
H = 128

NW = 32
NPAD = 10240
EPAD = NPAD * K


def _sc_gather128(table, idx):
    B = idx.shape[0]
    rw = B // NW
    C = rw // 8

    @functools.partial(
        pl.kernel,
        mesh=plsc.VectorSubcoreMesh(core_axis_name="c", subcore_axis_name="s"),
        out_type=jax.ShapeDtypeStruct((B, 128), jnp.float32),
        scratch_types=[
            pltpu.VMEM((C,), jnp.int32),
            pltpu.VMEM((C, 128), jnp.float32),
            pltpu.SemaphoreType.DMA,
        ],
    )
    def k(table_hbm, idx_hbm, out_hbm, idx_v, rows_v, sem):
        wid = lax.axis_index("s") * 2 + lax.axis_index("c")
        base = wid * rw
        for c in range(rw // C):
            off = base + c * C
            pltpu.sync_copy(idx_hbm.at[pl.ds(off, C)], idx_v)
            pltpu.async_copy(table_hbm.at[idx_v], rows_v, sem).wait()
            pltpu.sync_copy(rows_v, out_hbm.at[pl.ds(off, C)])

    return k(table, idx)


def _proj_body(v_ref, w_ref, o_ref):
    o_ref[...] = jnp.dot(v_ref[...], w_ref[...], preferred_element_type=jnp.float32)


def _proj_tables(node_attr, W1):
    W1r = jnp.concatenate([W1[s * D:(s + 1) * D, :] for s in range(1 + T)], axis=1)
    return pl.pallas_call(
        _proj_body,
        out_shape=jax.ShapeDtypeStruct((N, (1 + T) * 64), jnp.float32),
    )(node_attr, W1r)


def _choose_body(base_ref, cand_ref, b1_ref, w2_ref, b2_ref, noise_ref, nbrs_ref,
                 par_ref, o_ref):
    xb = jnp.concatenate([base_ref[...]] * (2 * K), axis=1)
    hidden = (xb + cand_ref[...]) + b1_ref[...]
    logpm2 = jnp.dot(jnp.maximum(hidden, 0.0), w2_ref[...],
                     preferred_element_type=jnp.float32) + b2_ref[0, 0]
    logpm = jnp.where(par_ref[...] == 1, logpm2[:, K:2 * K], logpm2[:, :K])
    segmax = jnp.max(logpm, axis=1, keepdims=True)
    e = jnp.exp(logpm - segmax)
    sums = jnp.sum(e, axis=1, keepdims=True)
    norm = segmax + jnp.log(sums)
    wp = jnp.exp(logpm - norm)
    wpn = wp + EPS * noise_ref[...]
    arg = jnp.argmax(wpn, axis=1)
    ii = jax.lax.broadcasted_iota(jnp.int32, wpn.shape, 1)
    sel = ii == arg[:, None]
    o_ref[...] = jnp.sum(jnp.where(sel, nbrs_ref[...], 0), axis=1, keepdims=True)


def _choose(base, cand2, b1t, W2big, b2, noise, nbrs, par):
    R = 1000
    return pl.pallas_call(
        _choose_body,
        grid=(N // R,),
        in_specs=[
            pl.BlockSpec((R, 64), lambda i: (i, 0)),
            pl.BlockSpec((R, K * 128), lambda i: (i, 0)),
            pl.BlockSpec((1, K * 128), lambda i: (0, 0)),
            pl.BlockSpec((K * 128, 2 * K), lambda i: (0, 0)),
            pl.BlockSpec((1, 1), lambda i: (0, 0)),
            pl.BlockSpec((R, K), lambda i: (i, 0)),
            pl.BlockSpec((R, K), lambda i: (i, 0)),
            pl.BlockSpec((R, K), lambda i: (i, 0)),
        ],
        out_specs=pl.BlockSpec((R, 1), lambda i: (i, 0)),
        out_shape=jax.ShapeDtypeStruct((N, 1), jnp.int32),
    )(base, cand2, b1t, W2big, b2.reshape(1, 1), noise, nbrs, par)


def _gru_body(x0_ref, x1_ref, x2_ref, x3_ref, wih_ref, whh_ref, bih_ref, bhh_ref,
              wout_ref, bout_ref, o_ref):
    xs = (x0_ref, x1_ref, x2_ref, x3_ref)
    h = jnp.zeros((x0_ref.shape[0], H), dtype=jnp.float32)
    for t in range(1 + T):
        gi = jnp.dot(xs[t][...], wih_ref[...], preferred_element_type=jnp.float32) + bih_ref[...]
        gh = jnp.dot(h, whh_ref[...], preferred_element_type=jnp.float32) + bhh_ref[...]
        ir, iz, inn = gi[:, :H], gi[:, H:2 * H], gi[:, 2 * H:]
        hr, hz, hn = gh[:, :H], gh[:, H:2 * H], gh[:, 2 * H:]
        r = jax.nn.sigmoid(ir + hr)
        z = jax.nn.sigmoid(iz + hz)
        n = jnp.tanh(inn + r * hn)
        h = (1.0 - z) * n + z * h
    o_ref[...] = jnp.dot(h, wout_ref[...], preferred_element_type=jnp.float32) + bout_ref[...]


def _gru_out(xs, W_ih, W_hh, b_ih, b_hh, W_out, b_out):
    R = 2000
    xspec = pl.BlockSpec((R, D), lambda i: (i, 0))
    return pl.pallas_call(
        _gru_body,
        grid=(N // R,),
        in_specs=[
            xspec, xspec, xspec, xspec,
            pl.BlockSpec((D, 3 * H), lambda i: (0, 0)),
            pl.BlockSpec((H, 3 * H), lambda i: (0, 0)),
            pl.BlockSpec((1, 3 * H), lambda i: (0, 0)),
            pl.BlockSpec((1, 3 * H), lambda i: (0, 0)),
            pl.BlockSpec((H, H), lambda i: (0, 0)),
            pl.BlockSpec((1, H), lambda i: (0, 0)),
        ],
        out_specs=pl.BlockSpec((R, H), lambda i: (i, 0)),
        out_shape=jax.ShapeDtypeStruct((N, H), jnp.float32),
    )(*xs, W_ih, W_hh, b_ih.reshape(1, -1), b_hh.reshape(1, -1), W_out, b_out.reshape(1, -1))


def kernel(node_attr, edge_index, slices, W1, b1, W2, b2, W_ih, W_hh, b_ih, b_hh, W_out, b_out):
    v = node_attr
    num_nodes = v.shape[0]
    proj_all = _proj_tables(v, W1)
    proj = [proj_all[:, s * 64:(s + 1) * 64] for s in range(1 + T)]
    projpk = [p.reshape(N // 2, 128) for p in proj]
    b1t = jnp.tile(b1, 2 * K).reshape(1, K * 128)
    W2big = jnp.zeros((K * 128, 2 * K), jnp.float32)
    for j in range(K):
        W2big = W2big.at[j * 128:j * 128 + 64, j].set(W2[:, 0])
        W2big = W2big.at[j * 128 + 64:(j + 1) * 128, K + j].set(W2[:, 0])
    edge_dst = edge_index[1]
    lastp = jnp.arange(NPAD, dtype=jnp.int32) % num_nodes
    base = proj[0]
    xs = [v]
    key = jax.random.key(42)
    for t in range(T):
        starts = slices[lastp, 0]
        col_idx = (starts[:, None] + jnp.arange(K)[None, :]).reshape(-1)
        adj_pad = edge_dst[col_idx]
        cand2 = _sc_gather128(projpk[1 + t], adj_pad >> 1).reshape(NPAD, K * 128)
        key, sub = jax.random.split(key)
        noise = jax.random.normal(sub, (num_nodes * K,), dtype=jnp.float32).reshape(num_nodes, K)
        walks_t = _choose(base, cand2, b1t, W2big, b2, noise,
                          adj_pad.reshape(NPAD, K),
                          (adj_pad & 1).reshape(NPAD, K))[:, 0]
        lastp = jnp.concatenate([walks_t, jnp.zeros((NPAD - num_nodes,), jnp.int32)])
        xs.append(v[walks_t, :])
        if t < T - 1:
            base = base + proj[1 + t][walks_t]
    return _gru_out(xs, W_ih, W_hh, b_ih, b_hh, W_out, b_out)


# speedup vs baseline: 1.0095x; 1.0095x over previous
import functools

import jax
import jax.numpy as jnp
from jax import lax
from jax.experimental import pallas as pl
from jax.experimental.pallas import tpu as pltpu
from jax.experimental.pallas import tpu_sc as plsc

N = 10000
K = 16
D = 128
T = 3
EPS = 0.01
H = 128

NW = 32
NPAD = 10240
EPAD = NPAD * K


def _sc_gather128(table, idx):
    B = idx.shape[0]
    rw = B // NW
    C = rw // 16
    nchunks = rw // C

    @functools.partial(
        pl.kernel,
        mesh=plsc.VectorSubcoreMesh(core_axis_name="c", subcore_axis_name="s"),
        out_type=jax.ShapeDtypeStruct((B, 128), jnp.float32),
        scratch_types=[
            pltpu.VMEM((C,), jnp.int32),
            pltpu.VMEM((C,), jnp.int32),
            pltpu.VMEM((C, 128), jnp.float32),
            pltpu.VMEM((C, 128), jnp.float32),
            pltpu.SemaphoreType.DMA,
            pltpu.SemaphoreType.DMA,
        ],
    )
    def k(table_hbm, idx_hbm, out_hbm, idx_v0, idx_v1, rows_v0, rows_v1, sem0, sem1):
        wid = lax.axis_index("s") * 2 + lax.axis_index("c")
        base = wid * rw
        idxs = (idx_v0, idx_v1)
        rows = (rows_v0, rows_v1)
        sems = (sem0, sem1)
        prev = None
        for c in range(nchunks):
            b = c % 2
            off = base + c * C
            pltpu.sync_copy(idx_hbm.at[pl.ds(off, C)], idxs[b])
            cur = pltpu.async_copy(table_hbm.at[idxs[b]], rows[b], sems[b])
            if prev is not None:
                prev.wait()
                pltpu.sync_copy(rows[1 - b], out_hbm.at[pl.ds(off - C, C)])
            prev = cur
        prev.wait()
        pltpu.sync_copy(rows[(nchunks - 1) % 2],
                        out_hbm.at[pl.ds(base + (nchunks - 1) * C, C)])

    return k(table, idx)


def _proj_body(v_ref, w_ref, o_ref):
    o_ref[...] = jnp.dot(v_ref[...], w_ref[...], preferred_element_type=jnp.float32)


def _proj_tables(node_attr, W1):
    W1r = jnp.concatenate([W1[s * D:(s + 1) * D, :] for s in range(1 + T)], axis=1)
    return pl.pallas_call(
        _proj_body,
        out_shape=jax.ShapeDtypeStruct((N, (1 + T) * 64), jnp.float32),
    )(node_attr, W1r)


def _choose_body(base_ref, cand_ref, b1_ref, w2_ref, b2_ref, noise_ref, nbrs_ref,
                 par_ref, o_ref):
    xb = jnp.concatenate([base_ref[...]] * (2 * K), axis=1)
    hidden = (xb + cand_ref[...]) + b1_ref[...]
    logpm2 = jnp.dot(jnp.maximum(hidden, 0.0), w2_ref[...],
                     preferred_element_type=jnp.float32) + b2_ref[0, 0]
    logpm = jnp.where(par_ref[...] == 1, logpm2[:, K:2 * K], logpm2[:, :K])
    segmax = jnp.max(logpm, axis=1, keepdims=True)
    e = jnp.exp(logpm - segmax)
    sums = jnp.sum(e, axis=1, keepdims=True)
    norm = segmax + jnp.log(sums)
    wp = jnp.exp(logpm - norm)
    wpn = wp + EPS * noise_ref[...]
    arg = jnp.argmax(wpn, axis=1)
    ii = jax.lax.broadcasted_iota(jnp.int32, wpn.shape, 1)
    sel = ii == arg[:, None]
    o_ref[...] = jnp.sum(jnp.where(sel, nbrs_ref[...], 0), axis=1, keepdims=True)


def _choose(base, cand2, b1t, W2big, b2, noise, nbrs, par):
    R = 1000
    return pl.pallas_call(
        _choose_body,
        grid=(N // R,),
        in_specs=[
            pl.BlockSpec((R, 64), lambda i: (i, 0)),
            pl.BlockSpec((R, K * 128), lambda i: (i, 0)),
            pl.BlockSpec((1, K * 128), lambda i: (0, 0)),
            pl.BlockSpec((K * 128, 2 * K), lambda i: (0, 0)),
            pl.BlockSpec((1, 1), lambda i: (0, 0)),
            pl.BlockSpec((R, K), lambda i: (i, 0)),
            pl.BlockSpec((R, K), lambda i: (i, 0)),
            pl.BlockSpec((R, K), lambda i: (i, 0)),
        ],
        out_specs=pl.BlockSpec((R, 1), lambda i: (i, 0)),
        out_shape=jax.ShapeDtypeStruct((N, 1), jnp.int32),
    )(base, cand2, b1t, W2big, b2.reshape(1, 1), noise, nbrs, par)


def _gru_body(x0_ref, x1_ref, x2_ref, x3_ref, wih_ref, whh_ref, bih_ref, bhh_ref,
              wout_ref, bout_ref, o_ref):
    xs = (x0_ref, x1_ref, x2_ref, x3_ref)
    h = jnp.zeros((x0_ref.shape[0], H), dtype=jnp.float32)
    for t in range(1 + T):
        gi = jnp.dot(xs[t][...], wih_ref[...], preferred_element_type=jnp.float32) + bih_ref[...]
        gh = jnp.dot(h, whh_ref[...], preferred_element_type=jnp.float32) + bhh_ref[...]
        ir, iz, inn = gi[:, :H], gi[:, H:2 * H], gi[:, 2 * H:]
        hr, hz, hn = gh[:, :H], gh[:, H:2 * H], gh[:, 2 * H:]
        r = jax.nn.sigmoid(ir + hr)
        z = jax.nn.sigmoid(iz + hz)
        n = jnp.tanh(inn + r * hn)
        h = (1.0 - z) * n + z * h
    o_ref[...] = jnp.dot(h, wout_ref[...], preferred_element_type=jnp.float32) + bout_ref[...]


def _gru_out(xs, W_ih, W_hh, b_ih, b_hh, W_out, b_out):
    R = 2000
    xspec = pl.BlockSpec((R, D), lambda i: (i, 0))
    return pl.pallas_call(
        _gru_body,
        grid=(N // R,),
        in_specs=[
            xspec, xspec, xspec, xspec,
            pl.BlockSpec((D, 3 * H), lambda i: (0, 0)),
            pl.BlockSpec((H, 3 * H), lambda i: (0, 0)),
            pl.BlockSpec((1, 3 * H), lambda i: (0, 0)),
            pl.BlockSpec((1, 3 * H), lambda i: (0, 0)),
            pl.BlockSpec((H, H), lambda i: (0, 0)),
            pl.BlockSpec((1, H), lambda i: (0, 0)),
        ],
        out_specs=pl.BlockSpec((R, H), lambda i: (i, 0)),
        out_shape=jax.ShapeDtypeStruct((N, H), jnp.float32),
    )(*xs, W_ih, W_hh, b_ih.reshape(1, -1), b_hh.reshape(1, -1), W_out, b_out.reshape(1, -1))


def kernel(node_attr, edge_index, slices, W1, b1, W2, b2, W_ih, W_hh, b_ih, b_hh, W_out, b_out):
    v = node_attr
    num_nodes = v.shape[0]
    proj_all = _proj_tables(v, W1)
    proj = [proj_all[:, s * 64:(s + 1) * 64] for s in range(1 + T)]
    projpk = [p.reshape(N // 2, 128) for p in proj]
    b1t = jnp.tile(b1, 2 * K).reshape(1, K * 128)
    W2big = jnp.zeros((K * 128, 2 * K), jnp.float32)
    for j in range(K):
        W2big = W2big.at[j * 128:j * 128 + 64, j].set(W2[:, 0])
        W2big = W2big.at[j * 128 + 64:(j + 1) * 128, K + j].set(W2[:, 0])
    edge_dst = edge_index[1]
    lastp = jnp.arange(NPAD, dtype=jnp.int32) % num_nodes
    base = proj[0]
    xs = [v]
    key = jax.random.key(42)
    for t in range(T):
        starts = slices[lastp, 0]
        col_idx = (starts[:, None] + jnp.arange(K)[None, :]).reshape(-1)
        adj_pad = edge_dst[col_idx]
        cand2 = _sc_gather128(projpk[1 + t], adj_pad >> 1).reshape(NPAD, K * 128)
        key, sub = jax.random.split(key)
        noise = jax.random.normal(sub, (num_nodes * K,), dtype=jnp.float32).reshape(num_nodes, K)
        walks_t = _choose(base, cand2, b1t, W2big, b2, noise,
                          adj_pad.reshape(NPAD, K),
                          (adj_pad & 1).reshape(NPAD, K))[:, 0]
        lastp = jnp.concatenate([walks_t, jnp.zeros((NPAD - num_nodes,), jnp.int32)])
        xs.append(v[walks_t, :])
        if t < T - 1:
            base = base + proj[1 + t][walks_t]
    return _gru_out(xs, W_ih, W_hh, b_ih, b_hh, W_out, b_out)


# speedup vs baseline: 1.5585x; 1.5439x over previous
import functools

import jax
import jax.numpy as jnp
from jax import lax
from jax.experimental import pallas as pl
from jax.experimental.pallas import tpu as pltpu
from jax.experimental.pallas import tpu_sc as plsc

N = 10000
K = 16
D = 128
T = 3
EPS = 0.01
H = 128

NW = 32
NPAD = 10240
EPAD = NPAD * K


def _sc_gather64(table, idx):
    B = idx.shape[0]
    rw = B // NW
    C = rw // 16
    nchunks = rw // C

    @functools.partial(
        pl.kernel,
        mesh=plsc.VectorSubcoreMesh(core_axis_name="c", subcore_axis_name="s"),
        out_type=jax.ShapeDtypeStruct((B, 64), jnp.float32),
        compiler_params=pltpu.CompilerParams(use_tc_tiling_on_sc=False),
        scratch_types=[
            pltpu.VMEM((C,), jnp.int32),
            pltpu.VMEM((C,), jnp.int32),
            pltpu.VMEM((C, 64), jnp.float32),
            pltpu.VMEM((C, 64), jnp.float32),
            pltpu.SemaphoreType.DMA,
            pltpu.SemaphoreType.DMA,
        ],
    )
    def k(table_hbm, idx_hbm, out_hbm, idx_v0, idx_v1, rows_v0, rows_v1, sem0, sem1):
        wid = lax.axis_index("s") * 2 + lax.axis_index("c")
        base = wid * rw
        idxs = (idx_v0, idx_v1)
        rows = (rows_v0, rows_v1)
        sems = (sem0, sem1)
        prev = None
        for c in range(nchunks):
            b = c % 2
            off = base + c * C
            pltpu.sync_copy(idx_hbm.at[pl.ds(off, C)], idxs[b])
            cur = pltpu.async_copy(table_hbm.at[idxs[b]], rows[b], sems[b])
            if prev is not None:
                prev.wait()
                pltpu.sync_copy(rows[1 - b], out_hbm.at[pl.ds(off - C, C)])
            prev = cur
        prev.wait()
        pltpu.sync_copy(rows[(nchunks - 1) % 2],
                        out_hbm.at[pl.ds(base + (nchunks - 1) * C, C)])

    return k(table, idx)


def _proj_body(v_ref, w_ref, o_ref):
    o_ref[...] = jnp.dot(v_ref[...], w_ref[...], preferred_element_type=jnp.float32)


def _proj_tables(node_attr, W1):
    W1r = jnp.concatenate([W1[s * D:(s + 1) * D, :] for s in range(1 + T)], axis=1)
    return pl.pallas_call(
        _proj_body,
        out_shape=jax.ShapeDtypeStruct((N, (1 + T) * 64), jnp.float32),
    )(node_attr, W1r)


def _choose_body(base_ref, cand_ref, b1_ref, w2_ref, b2_ref, noise_ref, nbrs_ref,
                 o_ref):
    xb = jnp.concatenate([base_ref[...]] * K, axis=1)
    hidden = (xb + cand_ref[...]) + b1_ref[...]
    logpm = jnp.dot(jnp.maximum(hidden, 0.0), w2_ref[...],
                    preferred_element_type=jnp.float32) + b2_ref[0, 0]
    segmax = jnp.max(logpm, axis=1, keepdims=True)
    e = jnp.exp(logpm - segmax)
    sums = jnp.sum(e, axis=1, keepdims=True)
    norm = segmax + jnp.log(sums)
    wp = jnp.exp(logpm - norm)
    wpn = wp + EPS * noise_ref[...]
    arg = jnp.argmax(wpn, axis=1)
    ii = jax.lax.broadcasted_iota(jnp.int32, wpn.shape, 1)
    sel = ii == arg[:, None]
    o_ref[...] = jnp.sum(jnp.where(sel, nbrs_ref[...], 0), axis=1, keepdims=True)


def _choose(base, cand2, b1t, W2big, b2, noise, nbrs):
    R = 1000
    return pl.pallas_call(
        _choose_body,
        grid=(N // R,),
        in_specs=[
            pl.BlockSpec((R, 64), lambda i: (i, 0)),
            pl.BlockSpec((R, K * 64), lambda i: (i, 0)),
            pl.BlockSpec((1, K * 64), lambda i: (0, 0)),
            pl.BlockSpec((K * 64, K), lambda i: (0, 0)),
            pl.BlockSpec((1, 1), lambda i: (0, 0)),
            pl.BlockSpec((R, K), lambda i: (i, 0)),
            pl.BlockSpec((R, K), lambda i: (i, 0)),
        ],
        out_specs=pl.BlockSpec((R, 1), lambda i: (i, 0)),
        out_shape=jax.ShapeDtypeStruct((N, 1), jnp.int32),
    )(base, cand2, b1t, W2big, b2.reshape(1, 1), noise, nbrs)


def _gru_body(x0_ref, x1_ref, x2_ref, x3_ref, wih_ref, whh_ref, bih_ref, bhh_ref,
              wout_ref, bout_ref, o_ref):
    xs = (x0_ref, x1_ref, x2_ref, x3_ref)
    h = jnp.zeros((x0_ref.shape[0], H), dtype=jnp.float32)
    for t in range(1 + T):
        gi = jnp.dot(xs[t][...], wih_ref[...], preferred_element_type=jnp.float32) + bih_ref[...]
        gh = jnp.dot(h, whh_ref[...], preferred_element_type=jnp.float32) + bhh_ref[...]
        ir, iz, inn = gi[:, :H], gi[:, H:2 * H], gi[:, 2 * H:]
        hr, hz, hn = gh[:, :H], gh[:, H:2 * H], gh[:, 2 * H:]
        r = jax.nn.sigmoid(ir + hr)
        z = jax.nn.sigmoid(iz + hz)
        n = jnp.tanh(inn + r * hn)
        h = (1.0 - z) * n + z * h
    o_ref[...] = jnp.dot(h, wout_ref[...], preferred_element_type=jnp.float32) + bout_ref[...]


def _gru_out(xs, W_ih, W_hh, b_ih, b_hh, W_out, b_out):
    R = 2000
    xspec = pl.BlockSpec((R, D), lambda i: (i, 0))
    return pl.pallas_call(
        _gru_body,
        grid=(N // R,),
        in_specs=[
            xspec, xspec, xspec, xspec,
            pl.BlockSpec((D, 3 * H), lambda i: (0, 0)),
            pl.BlockSpec((H, 3 * H), lambda i: (0, 0)),
            pl.BlockSpec((1, 3 * H), lambda i: (0, 0)),
            pl.BlockSpec((1, 3 * H), lambda i: (0, 0)),
            pl.BlockSpec((H, H), lambda i: (0, 0)),
            pl.BlockSpec((1, H), lambda i: (0, 0)),
        ],
        out_specs=pl.BlockSpec((R, H), lambda i: (i, 0)),
        out_shape=jax.ShapeDtypeStruct((N, H), jnp.float32),
    )(*xs, W_ih, W_hh, b_ih.reshape(1, -1), b_hh.reshape(1, -1), W_out, b_out.reshape(1, -1))


def kernel(node_attr, edge_index, slices, W1, b1, W2, b2, W_ih, W_hh, b_ih, b_hh, W_out, b_out):
    v = node_attr
    num_nodes = v.shape[0]
    proj_all = _proj_tables(v, W1)
    proj = [proj_all[:, s * 64:(s + 1) * 64] for s in range(1 + T)]
    b1t = jnp.tile(b1, K).reshape(1, K * 64)
    W2big = jnp.zeros((K * 64, K), jnp.float32)
    for j in range(K):
        W2big = W2big.at[j * 64:(j + 1) * 64, j].set(W2[:, 0])
    edge_dst = edge_index[1]
    lastp = jnp.arange(NPAD, dtype=jnp.int32) % num_nodes
    base = proj[0]
    xs = [v]
    key = jax.random.key(42)
    for t in range(T):
        starts = slices[lastp, 0]
        col_idx = (starts[:, None] + jnp.arange(K)[None, :]).reshape(-1)
        adj_pad = edge_dst[col_idx]
        cand2 = _sc_gather64(proj[1 + t], adj_pad).reshape(NPAD, K * 64)
        key, sub = jax.random.split(key)
        noise = jax.random.normal(sub, (num_nodes * K,), dtype=jnp.float32).reshape(num_nodes, K)
        walks_t = _choose(base, cand2, b1t, W2big, b2, noise,
                          adj_pad.reshape(NPAD, K))[:, 0]
        lastp = jnp.concatenate([walks_t, jnp.zeros((NPAD - num_nodes,), jnp.int32)])
        xs.append(v[walks_t, :])
        if t < T - 1:
            base = base + proj[1 + t][walks_t]
    return _gru_out(xs, W_ih, W_hh, b_ih, b_hh, W_out, b_out)
